# baseline (speedup 1.0000x reference)
import functools

import jax
import jax.numpy as jnp
from jax import lax
from jax.experimental import pallas as pl
from jax.experimental.pallas import tpu as pltpu
from jax.experimental.pallas import tpu_sc as plsc

ENT_N = 1000000
D = 64
BS = 4096
BSEQ = 8192
REG_C = 1e-05

NW = 32
E_IDX = 2 * BSEQ
E_PAD = E_IDX + 16
R_PAD = BSEQ + 16
DP = 128

CW = 128
NFULL = ENT_N // CW
TAILW = ENT_N - NFULL * CW
BASE_CH = NFULL // NW
EXTRA = NFULL - BASE_CH * NW
SCAN = 2048


@functools.cache
def _sc_sweep_fn():
    mesh = plsc.VectorSubcoreMesh(core_axis_name="c", subcore_axis_name="s")

    @functools.partial(
        pl.kernel,
        mesh=mesh,
        out_type=[
            jax.ShapeDtypeStruct((E_PAD, DP), jnp.float32),
            jax.ShapeDtypeStruct((E_PAD, DP), jnp.float32),
            jax.ShapeDtypeStruct((R_PAD, DP), jnp.float32),
            jax.ShapeDtypeStruct((R_PAD, DP), jnp.float32),
            jax.ShapeDtypeStruct((NW, 16), jnp.float32),
        ],
        scratch_types=[
            pltpu.VMEM((D, CW), jnp.float32), pltpu.VMEM((D, CW), jnp.float32),
            pltpu.VMEM((D, CW), jnp.float32), pltpu.VMEM((D, CW), jnp.float32),
            pltpu.VMEM((D, CW), jnp.float32), pltpu.VMEM((D, CW), jnp.float32),
            pltpu.VMEM((D, CW), jnp.float32), pltpu.VMEM((D, CW), jnp.float32),
            pltpu.VMEM((E_IDX + 16,), jnp.int32),
            pltpu.VMEM((BSEQ + 16,), jnp.int32),
            pltpu.VMEM((SCAN,), jnp.int32),
            pltpu.VMEM((16, DP), jnp.float32), pltpu.VMEM((16, DP), jnp.float32),
            pltpu.VMEM((16,), jnp.int32),
            pltpu.VMEM((16,), jnp.float32),
            pltpu.SemaphoreType.DMA,
            pltpu.SemaphoreType.DMA,
            pltpu.SemaphoreType.DMA,
        ],
        compiler_params=pltpu.CompilerParams(needs_layout_passes=False),
    )
    def _sc_sweep(idx_e_hbm, idx_r_hbm, entT, etrT, relT, rtrT,
                  ht_o, httr_o, r_o, rtr_o, norm_o,
                  ea0, ea1, ea2, ea3, eb0, eb1, eb2, eb3,
                  ce, cr, scanb, stg0, stg1, islot, accv,
                  semA, semB, semS):
        wid = lax.axis_index("s") * 2 + lax.axis_index("c")
        iota = lax.iota(jnp.int32, 16)
        cstart = wid * BASE_CH + jnp.minimum(wid, EXTRA)
        nch = BASE_CH + jnp.where(wid < EXTRA, 1, 0)
        lo = cstart * CW
        hi = (cstart + nch) * CW
        tabs = (entT, etrT, relT, rtrT)
        bufsA = (ea0, ea1, ea2, ea3)
        bufsB = (eb0, eb1, eb2, eb3)

        accv[...] = jnp.zeros((16,), jnp.float32)

        def scan_list(src, n, compact):
            def piece(p, cnt):
                pltpu.sync_copy(src.at[pl.ds(p * SCAN, SCAN)], scanb)

                def group(g, cnt):
                    v = scanb[pl.ds(g * 16, 16)]
                    m = (v >= lo) & (v < hi)
                    c = jnp.sum(m.astype(jnp.int32))

                    @pl.when(c > 0)
                    def _():
                        bpos = iota + (p * SCAN + g * 16)
                        packed = ((v - lo) << 14) | bpos
                        plsc.store_compressed(compact.at[pl.ds(cnt, 16)],
                                              packed, mask=m)

                    return cnt + c

                return lax.fori_loop(0, SCAN // 16, group, cnt)

            return lax.fori_loop(0, n // SCAN, piece, 0)

        cnt_e = scan_list(idx_e_hbm, E_IDX, ce)
        cnt_r = scan_list(idx_r_hbm, BSEQ, cr)

        def start(bufs, sem, kk):
            base = (cstart + kk) * CW
            for t in range(4):
                pltpu.async_copy(tabs[t].at[:, pl.ds(base, CW)], bufs[t], sem)

        def drain(bufs, sem):
            for t in range(4):
                pltpu.make_async_copy(tabs[t].at[:, pl.ds(0, CW)],
                                      bufs[t], sem).wait()

        def add_norm(bufs, width):
            def d_body(d, a):
                for buf in bufs:
                    for c in range(width // 16):
                        a = a + jnp.abs(buf[d, pl.ds(c * 16, 16)])
                return a

            accv[...] = lax.fori_loop(0, D, d_body, accv[...])

        def serve(compact, cnt, local_base, width, pair, outs, dummy):
            ngroups = (cnt + 15) >> 4

            def group(g, _):
                pv = compact[pl.ds(g * 16, 16)]
                pos_ok = (iota + g * 16) < cnt
                local = pv >> 14
                m = pos_ok & (local >= local_base) & (local < local_base + width)
                c = jnp.sum(m.astype(jnp.int32))

                @pl.when(c > 0)
                def _():
                    bpos = pv & 16383
                    islot[...] = jnp.where(m, bpos, dummy)
                    for l in range(16):
                        pvl = pv[l]
                        ll = pvl >> 14
                        okl = ((g * 16 + l < cnt) & (ll >= local_base)
                               & (ll < local_base + width))

                        @pl.when(okl)
                        def _():
                            jv = jnp.full((16,), ll - local_base, jnp.int32)
                            for kk in range(4):
                                rv = iota + kk * 16
                                stg0[l, pl.ds(kk * 16, 16)] = (
                                    plsc.load_gather(pair[0], [rv, jv]))
                                stg1[l, pl.ds(kk * 16, 16)] = (
                                    plsc.load_gather(pair[1], [rv, jv]))
                    pltpu.async_copy(stg0, outs[0].at[islot], semS).wait()
                    pltpu.async_copy(stg1, outs[1].at[islot], semS).wait()

                return 0

            lax.fori_loop(0, ngroups, group, 0)

        def process(bufs, k, width):
            add_norm(bufs, width)
            local_base = k * CW
            serve(ce, cnt_e, local_base, width, (bufs[0], bufs[1]),
                  (ht_o, httr_o), E_IDX)
            serve(cr, cnt_r, local_base, width, (bufs[2], bufs[3]),
                  (r_o, rtr_o), BSEQ)

        start(bufsA, semA, 0)

        def body(k, _):
            @pl.when((k & 1) == 0)
            def _():
                drain(bufsA, semA)

                @pl.when(k + 1 < nch)
                def _():
                    start(bufsB, semB, k + 1)

                process(bufsA, k, CW)

            @pl.when((k & 1) == 1)
            def _():
                drain(bufsB, semB)

                @pl.when(k + 1 < nch)
                def _():
                    start(bufsA, semA, k + 1)

                process(bufsB, k, CW)

            return 0

        lax.fori_loop(0, nch, body, 0)

        pltpu.sync_copy(accv, norm_o.at[wid])

    return _sc_sweep


TB = NFULL * CW
P = 1024
NLG = BS // P


def _loss_body(hp_r, hn_r, tp_r, tn_r, hpt_r, hnt_r, tpt_r, tnt_r,
               rp_r, rn_r, rtp_r, rtn_r,
               ohp_r, ohn_r, otp_r, otn_r, orp_r, orn_r,
               te_ref, tetr_ref, trl_ref, trtr_ref, np_ref, out_ref):
    i = pl.program_id(0)

    def patch(rows_ref, oh_ref, tail_ref):
        oh = oh_ref[...]
        fix = jax.lax.dot(oh, tail_ref[...],
                          precision=jax.lax.Precision.HIGHEST)
        it = jnp.sum(oh, axis=1, keepdims=True) > 0.5
        return jnp.where(it, fix, rows_ref[...])

    hp = patch(hp_r, ohp_r, te_ref)
    hn = patch(hn_r, ohn_r, te_ref)
    tp = patch(tp_r, otp_r, te_ref)
    tn = patch(tn_r, otn_r, te_ref)
    hpt = patch(hpt_r, ohp_r, tetr_ref)
    hnt = patch(hnt_r, ohn_r, tetr_ref)
    tpt = patch(tpt_r, otp_r, tetr_ref)
    tnt = patch(tnt_r, otn_r, tetr_ref)
    rp = patch(rp_r, orp_r, trl_ref)
    rn = patch(rn_r, orn_r, trl_ref)
    rtp = patch(rtp_r, orp_r, trtr_ref)
    rtn = patch(rtn_r, orn_r, trtr_ref)

    def transfer(e, etr, rtr):
        dot = jnp.sum(e * etr, axis=1, keepdims=True)
        e2 = e + dot * rtr
        n = jnp.sqrt(jnp.sum(e2 * e2, axis=1, keepdims=True))
        return e2 / jnp.maximum(n, 1e-12)

    a_p = jnp.abs(transfer(hp, hpt, rtp) + rp - transfer(tp, tpt, rtp) + 1e-06)
    a_n = jnp.abs(transfer(hn, hnt, rtn) + rn - transfer(tn, tnt, rtn) + 1e-06)
    rows = jnp.sum(a_p - a_n, axis=1, keepdims=True)
    hinge = jnp.sum(jnp.maximum(rows + 1.0, 0.0)) * (1.0 / BS)

    @pl.when(i == 0)
    def _():
        tail_norm = (jnp.sum(jnp.abs(te_ref[...]))
                     + jnp.sum(jnp.abs(tetr_ref[...]))
                     + jnp.sum(jnp.abs(trl_ref[...]))
                     + jnp.sum(jnp.abs(trtr_ref[...])))
        norm = (jnp.sum(np_ref[...]) + tail_norm) * (1.0 / ENT_N)
        out_ref[0, 0] = norm * REG_C

    out_ref[0, 0] += hinge


def _pb(off):
    return pl.BlockSpec((P, D), lambda i, off=off: (i + off, 0))


def _poh(off):
    return pl.BlockSpec((P, TAILW), lambda i, off=off: (i + off, 0))


_loss_call = pl.pallas_call(
    _loss_body,
    grid=(NLG,),
    in_specs=[
        _pb(0), _pb(NLG), _pb(2 * NLG), _pb(3 * NLG),
        _pb(0), _pb(NLG), _pb(2 * NLG), _pb(3 * NLG),
        _pb(0), _pb(NLG),
        _pb(0), _pb(NLG),
        _poh(0), _poh(NLG), _poh(2 * NLG), _poh(3 * NLG),
        _poh(0), _poh(NLG),
        pl.BlockSpec((TAILW, D), lambda i: (0, 0)),
        pl.BlockSpec((TAILW, D), lambda i: (0, 0)),
        pl.BlockSpec((TAILW, D), lambda i: (0, 0)),
        pl.BlockSpec((TAILW, D), lambda i: (0, 0)),
        pl.BlockSpec((NW, 16), lambda i: (0, 0)),
    ],
    out_specs=pl.BlockSpec((1, 1), lambda i: (0, 0), memory_space=pltpu.SMEM),
    out_shape=jax.ShapeDtypeStruct((1, 1), jnp.float32),
)


def kernel(input, ent_emb, rel_emb, ent_transfer, rel_transfer):
    idx_e = jnp.concatenate([input[:, 0], input[:, 2]])
    idx_r = input[:, 1]
    ht, httr, r, rtr, normp = _sc_sweep_fn()(
        idx_e, idx_r, ent_emb.T, ent_transfer.T, rel_emb.T, rel_transfer.T)
    tail_ids = TB + jnp.arange(TAILW, dtype=jnp.int32)
    oh_e = (idx_e[:, None] == tail_ids[None, :]).astype(jnp.float32)
    oh_r = (idx_r[:, None] == tail_ids[None, :]).astype(jnp.float32)
    ht64 = ht[0:E_IDX, 0:D]
    httr64 = httr[0:E_IDX, 0:D]
    r64 = r[0:BSEQ, 0:D]
    rtr64 = rtr[0:BSEQ, 0:D]
    out = _loss_call(ht64, ht64, ht64, ht64, httr64, httr64, httr64, httr64,
                     r64, r64, rtr64, rtr64,
                     oh_e, oh_e, oh_e, oh_e, oh_r, oh_r,
                     ent_emb[TB:ENT_N, :], ent_transfer[TB:ENT_N, :],
                     rel_emb[TB:ENT_N, :], rel_transfer[TB:ENT_N, :], normp)
    return out[0, 0]


# speedup vs baseline: 1.0014x; 1.0014x over previous
import functools

import jax
import jax.numpy as jnp
from jax import lax
from jax.experimental import pallas as pl
from jax.experimental.pallas import tpu as pltpu
from jax.experimental.pallas import tpu_sc as plsc

ENT_N = 1000000
D = 64
BS = 4096
BSEQ = 8192
REG_C = 1e-05

NW = 32
E_IDX = 2 * BSEQ
E_PAD = E_IDX + 16
R_PAD = BSEQ + 16
DP = 128

CW = 128
NFULL = ENT_N // CW
TAILW = ENT_N - NFULL * CW
BASE_CH = NFULL // NW
EXTRA = NFULL - BASE_CH * NW
SCAN = 2048


@functools.cache
def _sc_sweep_fn():
    mesh = plsc.VectorSubcoreMesh(core_axis_name="c", subcore_axis_name="s")

    @functools.partial(
        pl.kernel,
        mesh=mesh,
        out_type=[
            jax.ShapeDtypeStruct((E_PAD, DP), jnp.float32),
            jax.ShapeDtypeStruct((E_PAD, DP), jnp.float32),
            jax.ShapeDtypeStruct((R_PAD, DP), jnp.float32),
            jax.ShapeDtypeStruct((R_PAD, DP), jnp.float32),
            jax.ShapeDtypeStruct((NW, 16), jnp.float32),
        ],
        scratch_types=[
            pltpu.VMEM((D, CW), jnp.float32), pltpu.VMEM((D, CW), jnp.float32),
            pltpu.VMEM((D, CW), jnp.float32), pltpu.VMEM((D, CW), jnp.float32),
            pltpu.VMEM((D, CW), jnp.float32), pltpu.VMEM((D, CW), jnp.float32),
            pltpu.VMEM((D, CW), jnp.float32), pltpu.VMEM((D, CW), jnp.float32),
            pltpu.VMEM((E_IDX + 16,), jnp.int32),
            pltpu.VMEM((BSEQ + 16,), jnp.int32),
            pltpu.VMEM((SCAN,), jnp.int32),
            pltpu.VMEM((16, DP), jnp.float32), pltpu.VMEM((16, DP), jnp.float32),
            pltpu.VMEM((16,), jnp.int32),
            pltpu.VMEM((16,), jnp.float32),
            pltpu.SMEM((4,), jnp.int32),
            pltpu.SemaphoreType.DMA,
            pltpu.SemaphoreType.DMA,
            pltpu.SemaphoreType.DMA,
        ],
        compiler_params=pltpu.CompilerParams(needs_layout_passes=False),
    )
    def _sc_sweep(idx_e_hbm, idx_r_hbm, entT, etrT, relT, rtrT,
                  ht_o, httr_o, r_o, rtr_o, norm_o,
                  ea0, ea1, ea2, ea3, eb0, eb1, eb2, eb3,
                  ce, cr, scanb, stg0, stg1, islot, accv, smem,
                  semA, semB, semS):
        wid = lax.axis_index("s") * 2 + lax.axis_index("c")
        iota = lax.iota(jnp.int32, 16)
        cstart = wid * BASE_CH + jnp.minimum(wid, EXTRA)
        nch = BASE_CH + jnp.where(wid < EXTRA, 1, 0)
        lo = cstart * CW
        hi = (cstart + nch) * CW
        tabs = (entT, etrT, relT, rtrT)
        bufsA = (ea0, ea1, ea2, ea3)
        bufsB = (eb0, eb1, eb2, eb3)

        accv[...] = jnp.zeros((16,), jnp.float32)
        smem[0] = 0

        def scan_list(src, n, compact):
            def piece(p, cnt):
                pltpu.sync_copy(src.at[pl.ds(p * SCAN, SCAN)], scanb)

                def group(g, cnt):
                    v = scanb[pl.ds(g * 16, 16)]
                    m = (v >= lo) & (v < hi)
                    c = jnp.sum(m.astype(jnp.int32))

                    @pl.when(c > 0)
                    def _():
                        bpos = iota + (p * SCAN + g * 16)
                        packed = ((v - lo) << 14) | bpos
                        plsc.store_compressed(compact.at[pl.ds(cnt, 16)],
                                              packed, mask=m)

                    return cnt + c

                return lax.fori_loop(0, SCAN // 16, group, cnt)

            return lax.fori_loop(0, n // SCAN, piece, 0)

        cnt_e = scan_list(idx_e_hbm, E_IDX, ce)
        cnt_r = scan_list(idx_r_hbm, BSEQ, cr)

        def start(bufs, sem, kk):
            base = (cstart + kk) * CW
            for t in range(4):
                pltpu.async_copy(tabs[t].at[:, pl.ds(base, CW)], bufs[t], sem)

        def drain(bufs, sem):
            for t in range(4):
                pltpu.make_async_copy(tabs[t].at[:, pl.ds(0, CW)],
                                      bufs[t], sem).wait()

        def add_norm(bufs, width):
            def d_body(d, a):
                for buf in bufs:
                    for c in range(width // 16):
                        a = a + jnp.abs(buf[d, pl.ds(c * 16, 16)])
                return a

            accv[...] = lax.fori_loop(0, D, d_body, accv[...])

        def serve(compact, cnt, local_base, width, pair, outs, dummy):
            ngroups = (cnt + 15) >> 4

            def group(g, _):
                pv = compact[pl.ds(g * 16, 16)]
                pos_ok = (iota + g * 16) < cnt
                local = pv >> 14
                m = pos_ok & (local >= local_base) & (local < local_base + width)
                c = jnp.sum(m.astype(jnp.int32))

                @pl.when(c > 0)
                def _():
                    @pl.when(smem[0] > 0)
                    def _():
                        pltpu.make_async_copy(stg0, outs[0].at[islot], semS).wait()
                        pltpu.make_async_copy(stg1, outs[1].at[islot], semS).wait()

                    bpos = pv & 16383
                    islot[...] = jnp.where(m, bpos, dummy)
                    for l in range(16):
                        pvl = pv[l]
                        ll = pvl >> 14
                        okl = ((g * 16 + l < cnt) & (ll >= local_base)
                               & (ll < local_base + width))

                        @pl.when(okl)
                        def _():
                            jv = jnp.full((16,), ll - local_base, jnp.int32)
                            for kk in range(4):
                                rv = iota + kk * 16
                                stg0[l, pl.ds(kk * 16, 16)] = (
                                    plsc.load_gather(pair[0], [rv, jv]))
                                stg1[l, pl.ds(kk * 16, 16)] = (
                                    plsc.load_gather(pair[1], [rv, jv]))
                    pltpu.async_copy(stg0, outs[0].at[islot], semS)
                    pltpu.async_copy(stg1, outs[1].at[islot], semS)
                    smem[0] = 1

                return 0

            lax.fori_loop(0, ngroups, group, 0)

        def process(bufs, k, width):
            add_norm(bufs, width)
            local_base = k * CW
            serve(ce, cnt_e, local_base, width, (bufs[0], bufs[1]),
                  (ht_o, httr_o), E_IDX)
            serve(cr, cnt_r, local_base, width, (bufs[2], bufs[3]),
                  (r_o, rtr_o), BSEQ)

        start(bufsA, semA, 0)

        def body(k, _):
            @pl.when((k & 1) == 0)
            def _():
                drain(bufsA, semA)

                @pl.when(k + 1 < nch)
                def _():
                    start(bufsB, semB, k + 1)

                process(bufsA, k, CW)

            @pl.when((k & 1) == 1)
            def _():
                drain(bufsB, semB)

                @pl.when(k + 1 < nch)
                def _():
                    start(bufsA, semA, k + 1)

                process(bufsB, k, CW)

            return 0

        lax.fori_loop(0, nch, body, 0)

        @pl.when(smem[0] > 0)
        def _():
            pltpu.make_async_copy(stg0, ht_o.at[islot], semS).wait()
            pltpu.make_async_copy(stg1, ht_o.at[islot], semS).wait()

        pltpu.sync_copy(accv, norm_o.at[wid])

    return _sc_sweep


TB = NFULL * CW
P = 1024
NLG = BS // P


def _loss_body(hp_r, hn_r, tp_r, tn_r, hpt_r, hnt_r, tpt_r, tnt_r,
               rp_r, rn_r, rtp_r, rtn_r,
               ohp_r, ohn_r, otp_r, otn_r, orp_r, orn_r,
               te_ref, tetr_ref, trl_ref, trtr_ref, np_ref, out_ref):
    i = pl.program_id(0)

    def patch(rows_ref, oh_ref, tail_ref):
        oh = oh_ref[...]
        fix = jax.lax.dot(oh, tail_ref[...],
                          precision=jax.lax.Precision.HIGHEST)
        it = jnp.sum(oh, axis=1, keepdims=True) > 0.5
        return jnp.where(it, fix, rows_ref[...])

    hp = patch(hp_r, ohp_r, te_ref)
    hn = patch(hn_r, ohn_r, te_ref)
    tp = patch(tp_r, otp_r, te_ref)
    tn = patch(tn_r, otn_r, te_ref)
    hpt = patch(hpt_r, ohp_r, tetr_ref)
    hnt = patch(hnt_r, ohn_r, tetr_ref)
    tpt = patch(tpt_r, otp_r, tetr_ref)
    tnt = patch(tnt_r, otn_r, tetr_ref)
    rp = patch(rp_r, orp_r, trl_ref)
    rn = patch(rn_r, orn_r, trl_ref)
    rtp = patch(rtp_r, orp_r, trtr_ref)
    rtn = patch(rtn_r, orn_r, trtr_ref)

    def transfer(e, etr, rtr):
        dot = jnp.sum(e * etr, axis=1, keepdims=True)
        e2 = e + dot * rtr
        n = jnp.sqrt(jnp.sum(e2 * e2, axis=1, keepdims=True))
        return e2 / jnp.maximum(n, 1e-12)

    a_p = jnp.abs(transfer(hp, hpt, rtp) + rp - transfer(tp, tpt, rtp) + 1e-06)
    a_n = jnp.abs(transfer(hn, hnt, rtn) + rn - transfer(tn, tnt, rtn) + 1e-06)
    rows = jnp.sum(a_p - a_n, axis=1, keepdims=True)
    hinge = jnp.sum(jnp.maximum(rows + 1.0, 0.0)) * (1.0 / BS)

    @pl.when(i == 0)
    def _():
        tail_norm = (jnp.sum(jnp.abs(te_ref[...]))
                     + jnp.sum(jnp.abs(tetr_ref[...]))
                     + jnp.sum(jnp.abs(trl_ref[...]))
                     + jnp.sum(jnp.abs(trtr_ref[...])))
        norm = (jnp.sum(np_ref[...]) + tail_norm) * (1.0 / ENT_N)
        out_ref[0, 0] = norm * REG_C

    out_ref[0, 0] += hinge


def _pb(off):
    return pl.BlockSpec((P, D), lambda i, off=off: (i + off, 0))


def _poh(off):
    return pl.BlockSpec((P, TAILW), lambda i, off=off: (i + off, 0))


_loss_call = pl.pallas_call(
    _loss_body,
    grid=(NLG,),
    in_specs=[
        _pb(0), _pb(NLG), _pb(2 * NLG), _pb(3 * NLG),
        _pb(0), _pb(NLG), _pb(2 * NLG), _pb(3 * NLG),
        _pb(0), _pb(NLG),
        _pb(0), _pb(NLG),
        _poh(0), _poh(NLG), _poh(2 * NLG), _poh(3 * NLG),
        _poh(0), _poh(NLG),
        pl.BlockSpec((TAILW, D), lambda i: (0, 0)),
        pl.BlockSpec((TAILW, D), lambda i: (0, 0)),
        pl.BlockSpec((TAILW, D), lambda i: (0, 0)),
        pl.BlockSpec((TAILW, D), lambda i: (0, 0)),
        pl.BlockSpec((NW, 16), lambda i: (0, 0)),
    ],
    out_specs=pl.BlockSpec((1, 1), lambda i: (0, 0), memory_space=pltpu.SMEM),
    out_shape=jax.ShapeDtypeStruct((1, 1), jnp.float32),
)


def kernel(input, ent_emb, rel_emb, ent_transfer, rel_transfer):
    idx_e = jnp.concatenate([input[:, 0], input[:, 2]])
    idx_r = input[:, 1]
    ht, httr, r, rtr, normp = _sc_sweep_fn()(
        idx_e, idx_r, ent_emb.T, ent_transfer.T, rel_emb.T, rel_transfer.T)
    tail_ids = TB + jnp.arange(TAILW, dtype=jnp.int32)
    oh_e = (idx_e[:, None] == tail_ids[None, :]).astype(jnp.float32)
    oh_r = (idx_r[:, None] == tail_ids[None, :]).astype(jnp.float32)
    ht64 = ht[0:E_IDX, 0:D]
    httr64 = httr[0:E_IDX, 0:D]
    r64 = r[0:BSEQ, 0:D]
    rtr64 = rtr[0:BSEQ, 0:D]
    out = _loss_call(ht64, ht64, ht64, ht64, httr64, httr64, httr64, httr64,
                     r64, r64, rtr64, rtr64,
                     oh_e, oh_e, oh_e, oh_e, oh_r, oh_r,
                     ent_emb[TB:ENT_N, :], ent_transfer[TB:ENT_N, :],
                     rel_emb[TB:ENT_N, :], rel_transfer[TB:ENT_N, :], normp)
    return out[0, 0]


# speedup vs baseline: 4.4405x; 4.4343x over previous
import functools

import jax
import jax.numpy as jnp
from jax import lax
from jax.experimental import pallas as pl
from jax.experimental.pallas import tpu as pltpu
from jax.experimental.pallas import tpu_sc as plsc

ENT_N = 1000000
REL_N = 1000000
D = 64
BS = 4096
BSEQ = 8192
REG_C = 1e-05

NW = 32
E_IDX = 2 * BSEQ
E_PER = E_IDX // NW
R_PER = BSEQ // NW


GCHUNK = 128


@functools.cache
def _sc_gather_fn():
    mesh = plsc.VectorSubcoreMesh(core_axis_name="c", subcore_axis_name="s")

    @functools.partial(
        pl.kernel,
        mesh=mesh,
        out_type=[
            jax.ShapeDtypeStruct((E_IDX, D), jnp.float32),
            jax.ShapeDtypeStruct((E_IDX, D), jnp.float32),
            jax.ShapeDtypeStruct((BSEQ, D), jnp.float32),
            jax.ShapeDtypeStruct((BSEQ, D), jnp.float32),
        ],
        scratch_types=[
            pltpu.VMEM((E_PER,), jnp.int32),
            pltpu.VMEM((R_PER,), jnp.int32),
            pltpu.VMEM((E_PER, D), jnp.float32),
            pltpu.VMEM((E_PER, D), jnp.float32),
            pltpu.VMEM((R_PER, D), jnp.float32),
            pltpu.VMEM((R_PER, D), jnp.float32),
            pltpu.SemaphoreType.DMA,
        ],
        compiler_params=pltpu.CompilerParams(use_tc_tiling_on_sc=False),
    )
    def _sc_gather(idx_e_hbm, idx_r_hbm, ent_emb, ent_tr, rel_emb, rel_tr,
                   ht_out, httr_out, r_out, rtr_out,
                   idx_e_v, idx_r_v, rows_he, rows_htr, rows_r, rows_rtr, sem):
        wid = lax.axis_index("s") * 2 + lax.axis_index("c")
        be = wid * E_PER
        br = wid * R_PER
        pltpu.sync_copy(idx_e_hbm.at[pl.ds(be, E_PER)], idx_e_v)
        pltpu.sync_copy(idx_r_hbm.at[pl.ds(br, R_PER)], idx_r_v)
        copies = []
        for j in range(E_PER // GCHUNK):
            s = pl.ds(j * GCHUNK, GCHUNK)
            copies.append(pltpu.async_copy(ent_emb.at[idx_e_v.at[s]], rows_he.at[s], sem))
            copies.append(pltpu.async_copy(ent_tr.at[idx_e_v.at[s]], rows_htr.at[s], sem))
        for j in range(R_PER // GCHUNK):
            s = pl.ds(j * GCHUNK, GCHUNK)
            copies.append(pltpu.async_copy(rel_emb.at[idx_r_v.at[s]], rows_r.at[s], sem))
            copies.append(pltpu.async_copy(rel_tr.at[idx_r_v.at[s]], rows_rtr.at[s], sem))
        for c in copies:
            c.wait()
        pltpu.sync_copy(rows_he, ht_out.at[pl.ds(be, E_PER)])
        pltpu.sync_copy(rows_htr, httr_out.at[pl.ds(be, E_PER)])
        pltpu.sync_copy(rows_r, r_out.at[pl.ds(br, R_PER)])
        pltpu.sync_copy(rows_rtr, rtr_out.at[pl.ds(br, R_PER)])

    return _sc_gather


def _loss_body(ht_ref, httr_ref, r_ref, rtr_ref, out_ref):
    r_tr = rtr_ref[...]

    def transfer(e, etr):
        dot = jnp.sum(e * etr, axis=1, keepdims=True)
        e2 = e + dot * r_tr
        n = jnp.sqrt(jnp.sum(e2 * e2, axis=1, keepdims=True))
        return e2 / jnp.maximum(n, 1e-12)

    h = transfer(ht_ref[0:BSEQ, :], httr_ref[0:BSEQ, :])
    t = transfer(ht_ref[BSEQ:E_IDX, :], httr_ref[BSEQ:E_IDX, :])
    a = jnp.abs(h + r_ref[...] - t + 1e-06)
    diff = a[0:BS, :] - a[BS:BSEQ, :]
    rows = jnp.sum(diff, axis=1, keepdims=True)
    out_ref[0, 0] = jnp.sum(jnp.maximum(rows + 1.0, 0.0)) * (1.0 / BS)


_loss_call = pl.pallas_call(
    _loss_body,
    out_specs=pl.BlockSpec(memory_space=pltpu.SMEM),
    out_shape=jax.ShapeDtypeStruct((1, 1), jnp.float32),
)

NCHUNK = 16384
NGRID = -(-ENT_N // NCHUNK)


def _norm_body(a_ref, b_ref, c_ref, d_ref, out_ref):
    i = pl.program_id(0)
    rem = ENT_N - i * NCHUNK

    @pl.when(i == 0)
    def _():
        out_ref[0, 0] = 0.0

    @pl.when(rem >= NCHUNK)
    def _():
        s_ent = jnp.sum(jnp.abs(a_ref[...])) + jnp.sum(jnp.abs(c_ref[...]))
        s_rel = jnp.sum(jnp.abs(b_ref[...])) + jnp.sum(jnp.abs(d_ref[...]))
        out_ref[0, 0] += s_ent * (1.0 / ENT_N) + s_rel * (1.0 / REL_N)

    @pl.when(rem < NCHUNK)
    def _():
        m = jax.lax.broadcasted_iota(jnp.int32, (D, NCHUNK), 1) < rem

        def masked(ref):
            return jnp.sum(jnp.where(m, jnp.abs(ref[...]), 0.0))

        s_ent = masked(a_ref) + masked(c_ref)
        s_rel = masked(b_ref) + masked(d_ref)
        out_ref[0, 0] += s_ent * (1.0 / ENT_N) + s_rel * (1.0 / REL_N)


_norm_call = pl.pallas_call(
    _norm_body,
    grid=(NGRID,),
    in_specs=[pl.BlockSpec((D, NCHUNK), lambda i: (0, i))] * 4,
    out_specs=pl.BlockSpec(memory_space=pltpu.SMEM),
    out_shape=jax.ShapeDtypeStruct((1, 1), jnp.float32),
)


def kernel(input, ent_emb, rel_emb, ent_transfer, rel_transfer):
    idx_e = jnp.concatenate([input[:, 0], input[:, 2]])
    idx_r = input[:, 1]
    ht, httr, r, rtr_g = _sc_gather_fn()(idx_e, idx_r, ent_emb, ent_transfer,
                                         rel_emb, rel_transfer)
    loss = _loss_call(ht, httr, r, rtr_g)
    norm = _norm_call(ent_emb.T, rel_emb.T, ent_transfer.T, rel_transfer.T)
    return loss[0, 0] + norm[0, 0] * REG_C
